# baseline (speedup 1.0000x reference)
import functools

import jax
import jax.numpy as jnp
import numpy as np
from jax.experimental import pallas as pl
from jax.experimental.pallas import tpu as pltpu

_FOLD = np.array([[[1, 0, 0], [0, 1, 1]],
                  [[1, 1, 0], [0, 0, 1]]], dtype=np.float32)


def _fold_weights_t(w_oihw):
    fold = jnp.asarray(_FOLD)
    return jnp.einsum("apk,bql,oikl->abpqoi", fold, fold, w_oihw)


def _conv_body(x_ref, w_ref, b_ref, o_ref, *, H, W, Cin, Cout):
    M = H * W
    bias_v = b_ref[...]

    win = {}
    for q in range(3):
        for p in range(3):
            win[(p, q)] = x_ref[0, q, :, p * W:p * W + M]

    for a in range(2):
        accs = []
        for b in range(2):
            acc = None
            for dy in range(2):
                for dx in range(2):
                    d = jnp.dot(w_ref[a, b, dy, dx], win[(a + dy, b + dx)],
                                preferred_element_type=jnp.float32)
                    acc = d if acc is None else acc + d
            accs.append(acc + bias_v)
        o_ref[0, a] = pltpu.pack_elementwise(accs, packed_dtype=jnp.bfloat16)


def kernel(x_nchw, conv_weight_oihw, conv_bias):
    N, C, H, W = x_nchw.shape
    Cout = conv_weight_oihw.shape[0]
    M = H * W

    xpad = jnp.pad(x_nchw.astype(jnp.bfloat16),
                   ((0, 0), (0, 0), (1, 1), (1, 1)))
    slabs = jnp.stack([xpad[:, :, :, q:q + W] for q in range(3)],
                      axis=1).reshape(N, 3, C, (H + 2) * W)

    w_t = _fold_weights_t(conv_weight_oihw).astype(jnp.bfloat16)
    bias2 = conv_bias.reshape(Cout, 1).astype(jnp.float32)

    body = functools.partial(_conv_body, H=H, W=W, Cin=C, Cout=Cout)
    y_packed = pl.pallas_call(
        body,
        out_shape=jax.ShapeDtypeStruct((N, 2, Cout, M), jnp.int32),
        grid=(N,),
        in_specs=[
            pl.BlockSpec((1, 3, C, (H + 2) * W), lambda n: (n, 0, 0, 0)),
            pl.BlockSpec((2, 2, 2, 2, Cout, C), lambda n: (0, 0, 0, 0, 0, 0)),
            pl.BlockSpec((Cout, 1), lambda n: (0, 0)),
        ],
        out_specs=pl.BlockSpec((1, 2, Cout, M), lambda n: (n, 0, 0, 0)),
        compiler_params=pltpu.CompilerParams(
            dimension_semantics=("parallel",)),
        cost_estimate=pl.CostEstimate(
            flops=int(2 * 16 * N * M * C * Cout),
            transcendentals=0,
            bytes_accessed=int(N * C * (3 * (H + 2) * W * 2 + 2 * M * 4)),
        ),
    )(slabs, w_t, bias2)

    yb = jax.lax.bitcast_convert_type(y_packed, jnp.bfloat16)
    yb = yb.reshape(N, 2, Cout, H, W, 2)
    y = jnp.transpose(yb, (0, 2, 3, 1, 4, 5)).astype(jnp.float32)
    return y.reshape(N, Cout, 2 * H, 2 * W)
